# baseline (speedup 1.0000x reference)
import functools

import jax
import jax.numpy as jnp
from jax import lax
from jax.experimental import pallas as pl
from jax.experimental.pallas import tpu as pltpu
from jax.experimental.pallas import tpu_sc as plsc

N, M, D, P, K = 4096, 16384, 128, 3, 3
BM = 256

_NC, _NS = 2, 16
_NW = _NC * _NS
_ROWS = K * M
_RPW = _ROWS // _NW
_CHUNK = 128


def _topk_weights_block(q_ref, pos_t_ref, xs_ref, w2_ref, b_ref,
                        idx_ref, wn_ref, part_ref):
    q = q_ref[...]
    pos_t = pos_t_ref[...]

    q2 = q[:, 0:1] * q[:, 0:1] + q[:, 1:2] * q[:, 1:2] + q[:, 2:3] * q[:, 2:3]
    p2 = (pos_t[0:1, :] * pos_t[0:1, :] + pos_t[1:2, :] * pos_t[1:2, :]
          + pos_t[2:3, :] * pos_t[2:3, :])
    qp = jax.lax.dot_general(q, pos_t, (((1,), (0,)), ((), ())),
                             preferred_element_type=jnp.float32)
    d2 = q2 + p2 - 2.0 * qp
    d2 = jnp.maximum(d2, 0.0)

    big = jnp.float32(3.4e38)
    iota = jax.lax.broadcasted_iota(jnp.int32, (1, N), 1).astype(jnp.float32)
    nf = jnp.float32(N)

    dmins = []
    args = []
    dcur = d2
    for _ in range(K):
        mj = jnp.min(dcur, axis=1, keepdims=True)
        aj = jnp.min(jnp.where(dcur == mj, iota, nf), axis=1,
                     keepdims=True)
        oh = iota == aj
        dmins.append(mj)
        args.append(aj)
        dcur = jnp.where(oh, big, dcur)

    w0 = 1.0 / (dmins[0] + 1e-16)
    w1 = 1.0 / (dmins[1] + 1e-16)
    w2 = 1.0 / (dmins[2] + 1e-16)
    wsum = w0 + w1 + w2

    idx_ref[...] = jnp.concatenate(
        [a.astype(jnp.int32) for a in args], axis=1)
    wn_ref[...] = jnp.concatenate(
        [w0 / wsum, w1 / wsum, w2 / wsum], axis=1)

    part_ref[...] = (
        jax.lax.dot_general(xs_ref[...], w2_ref[...], (((1,), (0,)), ((), ())),
                            preferred_element_type=jnp.float32)
        + b_ref[...])


def _sc_gather(table_hbm, idx_hbm, out_hbm, idx_v, rows_v, sem):
    wid = lax.axis_index("s") * _NC + lax.axis_index("c")
    base = wid * _RPW
    for c in range(_RPW // _CHUNK):
        off = base + c * _CHUNK
        pltpu.sync_copy(idx_hbm.at[pl.ds(off, _CHUNK)], idx_v)
        pltpu.async_copy(table_hbm.at[idx_v], rows_v, sem).wait()
        pltpu.sync_copy(rows_v, out_hbm.at[pl.ds(off, _CHUNK)])


def _interp_mlp_block(f_ref, wn_ref, part_ref, w1_ref, o_ref):
    wn = wn_ref[...]
    x_interp = (wn[:, 0:1] * f_ref[:, 0, :]
                + wn[:, 1:2] * f_ref[:, 1, :]
                + wn[:, 2:3] * f_ref[:, 2, :])
    h = jax.lax.dot_general(x_interp, w1_ref[...], (((1,), (0,)), ((), ())),
                            preferred_element_type=jnp.float32) + part_ref[...]
    o_ref[...] = jnp.maximum(h, 0.0)


def kernel(x, pos, batch, x_skip, pos_skip, batch_skip, W, b):
    pos_t = pos.T
    W1 = W[:D, :]
    W2 = W[D:, :]
    b2 = b.reshape(1, D)

    grid = (M // BM,)
    idx, wn, part = pl.pallas_call(
        _topk_weights_block,
        grid=grid,
        in_specs=[
            pl.BlockSpec((BM, P), lambda i: (i, 0)),
            pl.BlockSpec((P, N), lambda i: (0, 0)),
            pl.BlockSpec((BM, D), lambda i: (i, 0)),
            pl.BlockSpec((D, D), lambda i: (0, 0)),
            pl.BlockSpec((1, D), lambda i: (0, 0)),
        ],
        out_specs=[
            pl.BlockSpec((BM, K), lambda i: (i, 0)),
            pl.BlockSpec((BM, K), lambda i: (i, 0)),
            pl.BlockSpec((BM, D), lambda i: (i, 0)),
        ],
        out_shape=[
            jax.ShapeDtypeStruct((M, K), jnp.int32),
            jax.ShapeDtypeStruct((M, K), jnp.float32),
            jax.ShapeDtypeStruct((M, D), jnp.float32),
        ],
    )(pos_skip, pos_t, x_skip, W2, b2)

    mesh = plsc.VectorSubcoreMesh(core_axis_name="c", subcore_axis_name="s")
    gather = functools.partial(
        pl.kernel, mesh=mesh,
        out_type=jax.ShapeDtypeStruct((_ROWS, D), jnp.float32),
        scratch_types=[
            pltpu.VMEM((_CHUNK,), jnp.int32),
            pltpu.VMEM((_CHUNK, D), jnp.float32),
            pltpu.SemaphoreType.DMA,
        ],
    )(_sc_gather)
    feats = gather(x, idx.reshape(_ROWS))

    out = pl.pallas_call(
        _interp_mlp_block,
        grid=grid,
        in_specs=[
            pl.BlockSpec((BM, K, D), lambda i: (i, 0, 0)),
            pl.BlockSpec((BM, K), lambda i: (i, 0)),
            pl.BlockSpec((BM, D), lambda i: (i, 0)),
            pl.BlockSpec((D, D), lambda i: (0, 0)),
        ],
        out_specs=pl.BlockSpec((BM, D), lambda i: (i, 0)),
        out_shape=jax.ShapeDtypeStruct((M, D), jnp.float32),
    )(feats.reshape(M, K, D), wn, part, W1)
    return out
